# baseline (speedup 1.0000x reference)
import functools

import jax
import jax.numpy as jnp
from jax import lax
from jax.experimental import pallas as pl
from jax.experimental.pallas import tpu as pltpu
from jax.experimental.pallas import tpu_sc as plsc

B = 4096
H = 50
V = 100000
D = 64

NC = 2
NS = 16
NW = NC * NS
RPW = B // NW
CR = 1
NCHK = RPW // CR
NBUF = 16
LA = 8
T = NCHK // NBUF

_mesh = plsc.VectorSubcoreMesh(core_axis_name="c", subcore_axis_name="s")


@functools.partial(
    pl.kernel,
    mesh=_mesh,
    out_type=jax.ShapeDtypeStruct((B, H, D), jnp.float32),
    scratch_types=[
        pltpu.VMEM((RPW, H), jnp.int32),
        pltpu.VMEM((NBUF, H, D), jnp.float32),
        pltpu.SemaphoreType.DMA,
        pltpu.SemaphoreType.DMA,
    ],
    compiler_params=pltpu.CompilerParams(
        use_tc_tiling_on_sc=False,
        needs_layout_passes=False,
        disable_bounds_checks=True,
    ),
)
def _emb_lookup(idx_hbm, w_hbm, out_hbm, idx_v, rows_v, gsem, ssem):
    wid = lax.axis_index("s") * NC + lax.axis_index("c")
    row0 = wid * RPW

    pltpu.sync_copy(idx_hbm.at[pl.ds(row0, RPW)], idx_v)

    def issue_gather(c, slot):
        pltpu.async_copy(w_hbm.at[idx_v.at[c]], rows_v.at[slot], gsem)

    def wait_gather(c, slot):
        pltpu.make_async_copy(
            w_hbm.at[idx_v.at[c]], rows_v.at[slot], gsem
        ).wait()

    def issue_store(c, slot):
        pltpu.async_copy(rows_v.at[slot], out_hbm.at[row0 + c], ssem)

    def wait_one_store():
        pltpu.make_async_copy(rows_v.at[0], out_hbm.at[row0], ssem).wait()

    def step(c, b, store_wait, issue):
        if store_wait:
            wait_one_store()
        if issue:
            issue_gather(c + LA, (b + LA) % NBUF)
        wait_gather(c, b)
        issue_store(c, b)

    for c in range(LA):
        issue_gather(c, c)

    for b in range(NBUF):
        step(b, b, store_wait=(b >= NBUF - LA), issue=True)

    def outer(t, carry):
        for b in range(NBUF):
            step(t * NBUF + b, b, store_wait=True, issue=True)
        return carry

    lax.fori_loop(1, T - 1, outer, 0)

    for b in range(NBUF):
        c = (T - 1) * NBUF + b
        step(c, b, store_wait=True, issue=(c + LA < NCHK))

    for _ in range(NBUF - LA):
        wait_one_store()


def kernel(input_ids, weight):
    return _emb_lookup(input_ids.astype(jnp.int32), weight)
